# baseline (speedup 1.0000x reference)
import jax
import jax.numpy as jnp
from jax.experimental import pallas as pl
from jax.experimental.pallas import tpu as pltpu


def _dma_copy_kernel(x_ref, o_ref, sem):
    cp = pltpu.make_async_copy(x_ref, o_ref, sem)
    cp.start()
    cp.wait()


def kernel(method_soft, label, features):
    del label, features
    return pl.pallas_call(
        _dma_copy_kernel,
        out_shape=jax.ShapeDtypeStruct(method_soft.shape, method_soft.dtype),
        in_specs=[pl.BlockSpec(memory_space=pl.ANY)],
        out_specs=pl.BlockSpec(memory_space=pl.ANY),
        scratch_shapes=[pltpu.SemaphoreType.DMA],
    )(method_soft)


# speedup vs baseline: 10.2544x; 10.2544x over previous
import jax
import jax.numpy as jnp
from jax.experimental import pallas as pl
from jax.experimental.pallas import tpu as pltpu


_BLOCK_ROWS = 1024


def _copy_kernel(x_ref, o_ref):
    o_ref[...] = x_ref[...]


def kernel(method_soft, label, features):
    del label, features
    n, d = method_soft.shape
    return pl.pallas_call(
        _copy_kernel,
        out_shape=jax.ShapeDtypeStruct(method_soft.shape, method_soft.dtype),
        grid=(n // _BLOCK_ROWS,),
        in_specs=[pl.BlockSpec((_BLOCK_ROWS, d), lambda i: (i, 0))],
        out_specs=pl.BlockSpec((_BLOCK_ROWS, d), lambda i: (i, 0)),
    )(method_soft)
